# baseline (speedup 1.0000x reference)
import jax
import jax.numpy as jnp
from jax import lax
from jax.experimental import pallas as pl
from jax.experimental.pallas import tpu as pltpu
from jax.experimental.pallas import tpu_sc as plsc

F32 = jnp.float32

_N = 100000
_E = 1600000
_G = 64

_CH = 8960
_NCHUNK = 12
_NPAD = _CH * _NCHUNK
_EPT = 50176
_EPAD = _EPT * 32
_ER = _EPAD // 128
_DR = 784

_BN = 2000
_NBLK = _N // _BN


def _agg_body(deg_mode, zs_hbm, src_hbm, dst_hbm, zeros_hbm, out_hbm,
              sbuf, dbuf, csrc, coff, rows, sstage, ostage, acc, sem_g, sem_s):
    cid = lax.axis_index("c")
    sid = lax.axis_index("s")

    def _drain(sem):
        pltpu.make_async_copy(zs_hbm.at[sstage.at[0]],
                              rows.at[pl.ds(0, 128)], sem).wait()

    if deg_mode:
        pltpu.sync_copy(zs_hbm, rows.at[pl.ds(0, 128)])

    def flush(b, g):
        slot = jnp.bitwise_and(g, 1)

        @pl.when(g >= 2)
        def _():
            _drain(sem_s)

        for q in range(8):
            if not deg_mode:
                sstage[0, pl.ds(q * 16, 16)] = csrc[pl.ds(b * 128 + q * 16, 16)]
            ostage[slot, pl.ds(q * 16, 16)] = coff[pl.ds(b * 128 + q * 16, 16)]

        if deg_mode:
            pltpu.async_copy(rows.at[pl.ds(0, 128)],
                             acc.at[ostage.at[slot]], sem_s, add=True)
        else:
            rslot = rows.at[pl.ds(slot * 128, 128)]
            pltpu.async_copy(zs_hbm.at[sstage.at[0]], rslot, sem_g)
            _drain(sem_g)
            pltpu.async_copy(rslot, acc.at[ostage.at[slot]], sem_s, add=True)
        return g + 1

    def chunk_body(kk, carry):
        chunk = 2 * kk + cid
        lo = chunk * _CH
        hi = lo + _CH
        pltpu.sync_copy(zeros_hbm.at[pl.ds(0, 560)],
                        acc.at[pl.ds(sid * 560, 560)])
        plsc.subcore_barrier()

        def block_body(u, st):
            cur, g = st
            r0 = sid * 784 + u * 16
            if not deg_mode:
                pltpu.sync_copy(src_hbm.at[pl.ds(r0, 16)], sbuf)
            pltpu.sync_copy(dst_hbm.at[pl.ds(r0, 16)], dbuf)
            for j in range(16):
                for q in range(8):
                    d = dbuf[j, pl.ds(q * 16, 16)]
                    mask = (d >= lo) & (d < hi)
                    plsc.store_compressed(coff.at[pl.ds(cur, 16)],
                                          d - lo, mask=mask)
                    if not deg_mode:
                        s = sbuf[j, pl.ds(q * 16, 16)]
                        plsc.store_compressed(csrc.at[pl.ds(cur, 16)],
                                              s, mask=mask)
                    cur = cur + jnp.sum(jnp.where(mask, 1, 0))
            nb = lax.shift_right_logical(cur, 7)
            g = lax.fori_loop(0, nb, flush, g)
            base = nb * 128
            for q in range(8):
                coff[pl.ds(q * 16, 16)] = coff[pl.ds(base + q * 16, 16)]
                if not deg_mode:
                    csrc[pl.ds(q * 16, 16)] = csrc[pl.ds(base + q * 16, 16)]
            return (cur - base, g)

        cur, g = lax.fori_loop(0, 49, block_body, (0, 0))
        pad_off = jnp.full((16,), _CH, jnp.int32)
        pad_src = jnp.zeros((16,), jnp.int32)
        for q in range(8):
            coff[pl.ds(cur + q * 16, 16)] = pad_off
            if not deg_mode:
                csrc[pl.ds(cur + q * 16, 16)] = pad_src
        nbf = lax.shift_right_logical(cur + 127, 7)
        g = lax.fori_loop(0, nbf, flush, g)

        @pl.when(g >= 1)
        def _():
            _drain(sem_s)

        @pl.when(g >= 2)
        def _():
            _drain(sem_s)

        plsc.subcore_barrier()
        r = sid * 560
        pltpu.sync_copy(acc.at[pl.ds(r, 560)],
                        out_hbm.at[pl.ds(lo + r, 560)])
        plsc.subcore_barrier()
        return carry

    lax.fori_loop(0, _NCHUNK // 2, chunk_body, 0)


def _aggregate(zs, src2d, dst2d, zeros840, deg_mode=False):
    import functools as _ft
    return pl.kernel(
        _ft.partial(_agg_body, deg_mode),
        out_type=jax.ShapeDtypeStruct((_NPAD, 128), F32),
        mesh=plsc.VectorSubcoreMesh(core_axis_name="c", subcore_axis_name="s"),
        compiler_params=pltpu.CompilerParams(needs_layout_passes=False),
        scratch_types=[
            pltpu.VMEM((16, 128), jnp.int32),
            pltpu.VMEM((16, 128), jnp.int32),
            pltpu.VMEM((2304,), jnp.int32),
            pltpu.VMEM((2304,), jnp.int32),
            pltpu.VMEM((256, 128), F32),
            pltpu.VMEM((1, 128), jnp.int32),
            pltpu.VMEM((2, 128), jnp.int32),
            pltpu.VMEM_SHARED((_CH + 16, 128), F32),
            pltpu.SemaphoreType.DMA,
            pltpu.SemaphoreType.DMA,
        ],
    )(zs, src2d, dst2d, zeros840)


def _tc_a_body(xp_ref, w1_ref, dega_ref, zs_ref, dinv_ref):
    dv = lax.rsqrt(dega_ref[:, 0:1] + 1.0)
    dinv_ref[...] = dv
    z = jnp.dot(xp_ref[...], w1_ref[...], preferred_element_type=F32)
    zs_ref[...] = z * dv


def _tc_a(xp, w1p, dega):
    return pl.pallas_call(
        _tc_a_body,
        grid=(_NBLK,),
        in_specs=[
            pl.BlockSpec((_BN, 128), lambda i: (i, 0)),
            pl.BlockSpec((128, 128), lambda i: (0, 0)),
            pl.BlockSpec((_BN, 128), lambda i: (i, 0)),
        ],
        out_specs=[
            pl.BlockSpec((_BN, 128), lambda i: (i, 0)),
            pl.BlockSpec((_BN, 1), lambda i: (i, 0)),
        ],
        out_shape=[
            jax.ShapeDtypeStruct((_N, 128), F32),
            jax.ShapeDtypeStruct((_N, 1), F32),
        ],
    )(xp, w1p, dega)


def _tc_b_body(agg_ref, zs_ref, dinv_ref, w_ref, b_ref, out_ref):
    dv = dinv_ref[...]
    h = jax.nn.relu(dv * (agg_ref[...] + zs_ref[...]) + b_ref[...])
    out_ref[...] = jnp.dot(h, w_ref[...], preferred_element_type=F32) * dv


def _tc_b(agg, zs, dinv, w, b2d):
    return pl.pallas_call(
        _tc_b_body,
        grid=(_NBLK,),
        in_specs=[
            pl.BlockSpec((_BN, 128), lambda i: (i, 0)),
            pl.BlockSpec((_BN, 128), lambda i: (i, 0)),
            pl.BlockSpec((_BN, 1), lambda i: (i, 0)),
            pl.BlockSpec((128, 128), lambda i: (0, 0)),
            pl.BlockSpec((1, 128), lambda i: (0, 0)),
        ],
        out_specs=pl.BlockSpec((_BN, 128), lambda i: (i, 0)),
        out_shape=jax.ShapeDtypeStruct((_N, 128), F32),
    )(agg, zs, dinv, w, b2d)


def _tc_c_body(agg_ref, zs_ref, dinv_ref, b3_ref, batch_ref,
               wfc1_ref, bfc1_ref, wfc2_ref, bfc2_ref, out_ref,
               sums, cnts):
    i = pl.program_id(0)

    @pl.when(i == 0)
    def _():
        sums[...] = jnp.zeros_like(sums)
        cnts[...] = jnp.zeros_like(cnts)

    dv = dinv_ref[...]
    h = jax.nn.relu(dv * (agg_ref[...] + zs_ref[...]) + b3_ref[...])
    bt = batch_ref[0, 0, :]
    gid = lax.broadcasted_iota(jnp.int32, (_G, _BN), 0)
    mask = (bt[None, :] == gid).astype(F32)
    sums[...] += jnp.dot(mask, h, preferred_element_type=F32)
    cnts[...] += jnp.sum(mask, axis=1, keepdims=True)

    @pl.when(i == _NBLK - 1)
    def _():
        g = sums[...] / jnp.maximum(cnts[...], 1.0)
        a = jax.nn.relu(
            jnp.dot(g, wfc1_ref[...], preferred_element_type=F32)
            + bfc1_ref[...])
        out_ref[...] = (
            jnp.dot(a, wfc2_ref[...], preferred_element_type=F32)
            + bfc2_ref[...])


def _tc_c(agg, zs, dinv, b3, batchr, wfc1, bfc1, wfc2, bfc2):
    return pl.pallas_call(
        _tc_c_body,
        grid=(_NBLK,),
        in_specs=[
            pl.BlockSpec((_BN, 128), lambda i: (i, 0)),
            pl.BlockSpec((_BN, 128), lambda i: (i, 0)),
            pl.BlockSpec((_BN, 1), lambda i: (i, 0)),
            pl.BlockSpec((1, 128), lambda i: (0, 0)),
            pl.BlockSpec((1, 1, _BN), lambda i: (i, 0, 0)),
            pl.BlockSpec((128, 64), lambda i: (0, 0)),
            pl.BlockSpec((1, 64), lambda i: (0, 0)),
            pl.BlockSpec((64, 10), lambda i: (0, 0)),
            pl.BlockSpec((1, 10), lambda i: (0, 0)),
        ],
        out_specs=pl.BlockSpec((_G, 10), lambda i: (0, 0)),
        out_shape=jax.ShapeDtypeStruct((_G, 10), F32),
        scratch_shapes=[
            pltpu.VMEM((_G, 128), F32),
            pltpu.VMEM((_G, 1), F32),
        ],
    )(agg, zs, dinv, b3, batchr, wfc1, bfc1, wfc2, bfc2)


def kernel(x, edge_index, batch, W1, b1, W2, b2, W3, b3,
           Wfc1, bfc1, Wfc2, bfc2):
    pad = _EPAD - _E
    src2d = jnp.pad(edge_index[0], (0, pad)).reshape(_ER, 128)
    dst2d = jnp.pad(edge_index[1], (0, pad),
                    constant_values=_N).reshape(_ER, 128)

    zeros840 = jnp.zeros((840, 128), F32)

    xp = jnp.pad(x, ((0, 0), (0, 128 - x.shape[1])))
    w1p = jnp.pad(W1, ((0, 128 - W1.shape[0]), (0, 128 - W1.shape[1])))
    w2p = jnp.pad(W2, ((0, 64), (0, 0)))
    b1p = jnp.pad(b1, (0, 64))

    onehot = jnp.zeros((128, 128), F32).at[:, 0].set(1.0)
    dega = _aggregate(onehot, src2d, dst2d, zeros840, deg_mode=True)
    zs1, dinv = _tc_a(xp, w1p, dega)
    agg1 = _aggregate(zs1, src2d, dst2d, zeros840)
    zs2 = _tc_b(agg1, zs1, dinv, w2p, b1p.reshape(1, -1))
    agg2 = _aggregate(zs2, src2d, dst2d, zeros840)
    zs3 = _tc_b(agg2, zs2, dinv, W3, b2.reshape(1, -1))
    agg3 = _aggregate(zs3, src2d, dst2d, zeros840)

    batchr = batch.reshape(_NBLK, 1, _BN)
    return _tc_c(agg3, zs3, dinv, b3.reshape(1, -1), batchr,
                 Wfc1, bfc1.reshape(1, -1), Wfc2, bfc2.reshape(1, -1))


# speedup vs baseline: 1.4168x; 1.4168x over previous
import jax
import jax.numpy as jnp
from jax import lax
from jax.experimental import pallas as pl
from jax.experimental.pallas import tpu as pltpu
from jax.experimental.pallas import tpu_sc as plsc

F32 = jnp.float32

_N = 100000
_E = 1600000
_G = 64

_CH = 8960
_NCHUNK = 12
_NPAD = _CH * _NCHUNK
_EPT = 50176
_EPAD = _EPT * 32
_ER = _EPAD // 128
_DR = 784

_BN = 2000
_NBLK = _N // _BN


def _agg_body(deg_mode, zs_hbm, src_hbm, dst_hbm, zeros_hbm, out_hbm,
              sbuf, dbuf, csrc, coff, rows, sstage, ostage, acc,
              sem_g, sem_s, sem_l):
    cid = lax.axis_index("c")
    sid = lax.axis_index("s")

    def _drain(sem):
        pltpu.make_async_copy(zs_hbm.at[sstage.at[0]],
                              rows.at[pl.ds(0, 128)], sem).wait()

    def _drain_l():
        pltpu.make_async_copy(dst_hbm.at[pl.ds(0, 16)],
                              dbuf.at[pl.ds(0, 16)], sem_l).wait()

    if deg_mode:
        pltpu.sync_copy(zs_hbm, rows.at[pl.ds(0, 128)])

    def flush(b, g):
        slot = jnp.bitwise_and(g, 1)
        oslot = jnp.bitwise_and(g + 1, 1)

        @pl.when(g >= 2)
        def _():
            _drain(sem_s)

        for q in range(8):
            if not deg_mode:
                sstage[slot, pl.ds(q * 16, 16)] = csrc[pl.ds(b * 128 + q * 16, 16)]
            ostage[slot, pl.ds(q * 16, 16)] = coff[pl.ds(b * 128 + q * 16, 16)]
        if deg_mode:
            pltpu.async_copy(rows.at[pl.ds(0, 128)],
                             acc.at[ostage.at[slot]], sem_s, add=True)
        else:
            pltpu.async_copy(zs_hbm.at[sstage.at[slot]],
                             rows.at[pl.ds(slot * 128, 128)], sem_g)

            @pl.when(g >= 1)
            def _():
                _drain(sem_g)
                pltpu.async_copy(rows.at[pl.ds(oslot * 128, 128)],
                                 acc.at[ostage.at[oslot]], sem_s, add=True)
        return g + 1

    def chunk_body(kk, carry):
        chunk = 2 * kk + cid
        lo = chunk * _CH
        hi = lo + _CH
        r0 = sid * 784
        if not deg_mode:
            pltpu.async_copy(src_hbm.at[pl.ds(r0, 16)],
                             sbuf.at[pl.ds(0, 16)], sem_l)
        pltpu.async_copy(dst_hbm.at[pl.ds(r0, 16)],
                         dbuf.at[pl.ds(0, 16)], sem_l)
        pltpu.sync_copy(zeros_hbm.at[pl.ds(0, 560)],
                        acc.at[pl.ds(sid * 560, 560)])
        plsc.subcore_barrier()

        def block_body(u, st):
            cur, g = st
            p16 = jnp.bitwise_and(u, 1) * 16
            np16 = jnp.bitwise_and(u + 1, 1) * 16
            if not deg_mode:
                _drain_l()
            _drain_l()

            @pl.when(u + 1 < 49)
            def _():
                r1 = sid * 784 + (u + 1) * 16
                if not deg_mode:
                    pltpu.async_copy(src_hbm.at[pl.ds(r1, 16)],
                                     sbuf.at[pl.ds(np16, 16)], sem_l)
                pltpu.async_copy(dst_hbm.at[pl.ds(r1, 16)],
                                 dbuf.at[pl.ds(np16, 16)], sem_l)

            for j in range(16):
                for q in range(8):
                    d = dbuf[p16 + j, pl.ds(q * 16, 16)]
                    mask = (d >= lo) & (d < hi)
                    plsc.store_compressed(coff.at[pl.ds(cur, 16)],
                                          d - lo, mask=mask)
                    if not deg_mode:
                        s = sbuf[p16 + j, pl.ds(q * 16, 16)]
                        plsc.store_compressed(csrc.at[pl.ds(cur, 16)],
                                              s, mask=mask)
                    cur = cur + jnp.sum(jnp.where(mask, 1, 0))
            nb = lax.shift_right_logical(cur, 7)
            g = lax.fori_loop(0, nb, flush, g)
            base = nb * 128
            for q in range(8):
                coff[pl.ds(q * 16, 16)] = coff[pl.ds(base + q * 16, 16)]
                if not deg_mode:
                    csrc[pl.ds(q * 16, 16)] = csrc[pl.ds(base + q * 16, 16)]
            return (cur - base, g)

        cur, g = lax.fori_loop(0, 49, block_body, (0, 0))
        pad_off = jnp.full((16,), _CH, jnp.int32)
        pad_src = jnp.zeros((16,), jnp.int32)
        for q in range(8):
            coff[pl.ds(cur + q * 16, 16)] = pad_off
            if not deg_mode:
                csrc[pl.ds(cur + q * 16, 16)] = pad_src
        nbf = lax.shift_right_logical(cur + 127, 7)
        g = lax.fori_loop(0, nbf, flush, g)

        if not deg_mode:
            @pl.when(g >= 1)
            def _():
                _drain(sem_g)
                lslot = jnp.bitwise_and(g + 1, 1)
                pltpu.async_copy(rows.at[pl.ds(lslot * 128, 128)],
                                 acc.at[ostage.at[lslot]], sem_s, add=True)

        @pl.when(g >= 1)
        def _():
            _drain(sem_s)

        @pl.when(g >= 2)
        def _():
            _drain(sem_s)

        plsc.subcore_barrier()
        r = sid * 560
        pltpu.sync_copy(acc.at[pl.ds(r, 560)],
                        out_hbm.at[pl.ds(lo + r, 560)])
        plsc.subcore_barrier()
        return carry

    lax.fori_loop(0, _NCHUNK // 2, chunk_body, 0)


def _aggregate(zs, src2d, dst2d, zeros840, deg_mode=False):
    import functools as _ft
    return pl.kernel(
        _ft.partial(_agg_body, deg_mode),
        out_type=jax.ShapeDtypeStruct((_NPAD, 128), F32),
        mesh=plsc.VectorSubcoreMesh(core_axis_name="c", subcore_axis_name="s"),
        compiler_params=pltpu.CompilerParams(needs_layout_passes=False),
        scratch_types=[
            pltpu.VMEM((32, 128), jnp.int32),
            pltpu.VMEM((32, 128), jnp.int32),
            pltpu.VMEM((2304,), jnp.int32),
            pltpu.VMEM((2304,), jnp.int32),
            pltpu.VMEM((256, 128), F32),
            pltpu.VMEM((2, 128), jnp.int32),
            pltpu.VMEM((2, 128), jnp.int32),
            pltpu.VMEM_SHARED((_CH + 16, 128), F32),
            pltpu.SemaphoreType.DMA,
            pltpu.SemaphoreType.DMA,
            pltpu.SemaphoreType.DMA,
        ],
    )(zs, src2d, dst2d, zeros840)


def _tc_a_body(xp_ref, w1_ref, dega_ref, zs_ref, dinv_ref):
    dv = lax.rsqrt(dega_ref[:, 0:1] + 1.0)
    dinv_ref[...] = dv
    z = jnp.dot(xp_ref[...], w1_ref[...], preferred_element_type=F32)
    zs_ref[...] = z * dv


def _tc_a(xp, w1p, dega):
    return pl.pallas_call(
        _tc_a_body,
        grid=(_NBLK,),
        in_specs=[
            pl.BlockSpec((_BN, 128), lambda i: (i, 0)),
            pl.BlockSpec((128, 128), lambda i: (0, 0)),
            pl.BlockSpec((_BN, 128), lambda i: (i, 0)),
        ],
        out_specs=[
            pl.BlockSpec((_BN, 128), lambda i: (i, 0)),
            pl.BlockSpec((_BN, 1), lambda i: (i, 0)),
        ],
        out_shape=[
            jax.ShapeDtypeStruct((_N, 128), F32),
            jax.ShapeDtypeStruct((_N, 1), F32),
        ],
    )(xp, w1p, dega)


def _tc_b_body(agg_ref, zs_ref, dinv_ref, w_ref, b_ref, out_ref):
    dv = dinv_ref[...]
    h = jax.nn.relu(dv * (agg_ref[...] + zs_ref[...]) + b_ref[...])
    out_ref[...] = jnp.dot(h, w_ref[...], preferred_element_type=F32) * dv


def _tc_b(agg, zs, dinv, w, b2d):
    return pl.pallas_call(
        _tc_b_body,
        grid=(_NBLK,),
        in_specs=[
            pl.BlockSpec((_BN, 128), lambda i: (i, 0)),
            pl.BlockSpec((_BN, 128), lambda i: (i, 0)),
            pl.BlockSpec((_BN, 1), lambda i: (i, 0)),
            pl.BlockSpec((128, 128), lambda i: (0, 0)),
            pl.BlockSpec((1, 128), lambda i: (0, 0)),
        ],
        out_specs=pl.BlockSpec((_BN, 128), lambda i: (i, 0)),
        out_shape=jax.ShapeDtypeStruct((_N, 128), F32),
    )(agg, zs, dinv, w, b2d)


def _tc_c_body(agg_ref, zs_ref, dinv_ref, b3_ref, batch_ref,
               wfc1_ref, bfc1_ref, wfc2_ref, bfc2_ref, out_ref,
               sums, cnts):
    i = pl.program_id(0)

    @pl.when(i == 0)
    def _():
        sums[...] = jnp.zeros_like(sums)
        cnts[...] = jnp.zeros_like(cnts)

    dv = dinv_ref[...]
    h = jax.nn.relu(dv * (agg_ref[...] + zs_ref[...]) + b3_ref[...])
    bt = batch_ref[0, 0, :]
    gid = lax.broadcasted_iota(jnp.int32, (_G, _BN), 0)
    mask = (bt[None, :] == gid).astype(F32)
    sums[...] += jnp.dot(mask, h, preferred_element_type=F32)
    cnts[...] += jnp.sum(mask, axis=1, keepdims=True)

    @pl.when(i == _NBLK - 1)
    def _():
        g = sums[...] / jnp.maximum(cnts[...], 1.0)
        a = jax.nn.relu(
            jnp.dot(g, wfc1_ref[...], preferred_element_type=F32)
            + bfc1_ref[...])
        out_ref[...] = (
            jnp.dot(a, wfc2_ref[...], preferred_element_type=F32)
            + bfc2_ref[...])


def _tc_c(agg, zs, dinv, b3, batchr, wfc1, bfc1, wfc2, bfc2):
    return pl.pallas_call(
        _tc_c_body,
        grid=(_NBLK,),
        in_specs=[
            pl.BlockSpec((_BN, 128), lambda i: (i, 0)),
            pl.BlockSpec((_BN, 128), lambda i: (i, 0)),
            pl.BlockSpec((_BN, 1), lambda i: (i, 0)),
            pl.BlockSpec((1, 128), lambda i: (0, 0)),
            pl.BlockSpec((1, 1, _BN), lambda i: (i, 0, 0)),
            pl.BlockSpec((128, 64), lambda i: (0, 0)),
            pl.BlockSpec((1, 64), lambda i: (0, 0)),
            pl.BlockSpec((64, 10), lambda i: (0, 0)),
            pl.BlockSpec((1, 10), lambda i: (0, 0)),
        ],
        out_specs=pl.BlockSpec((_G, 10), lambda i: (0, 0)),
        out_shape=jax.ShapeDtypeStruct((_G, 10), F32),
        scratch_shapes=[
            pltpu.VMEM((_G, 128), F32),
            pltpu.VMEM((_G, 1), F32),
        ],
    )(agg, zs, dinv, b3, batchr, wfc1, bfc1, wfc2, bfc2)


def kernel(x, edge_index, batch, W1, b1, W2, b2, W3, b3,
           Wfc1, bfc1, Wfc2, bfc2):
    pad = _EPAD - _E
    src2d = jnp.pad(edge_index[0], (0, pad)).reshape(_ER, 128)
    dst2d = jnp.pad(edge_index[1], (0, pad),
                    constant_values=_N).reshape(_ER, 128)

    zeros840 = jnp.zeros((840, 128), F32)

    xp = jnp.pad(x, ((0, 0), (0, 128 - x.shape[1])))
    w1p = jnp.pad(W1, ((0, 128 - W1.shape[0]), (0, 128 - W1.shape[1])))
    w2p = jnp.pad(W2, ((0, 64), (0, 0)))
    b1p = jnp.pad(b1, (0, 64))

    onehot = jnp.zeros((128, 128), F32).at[:, 0].set(1.0)
    dega = _aggregate(onehot, src2d, dst2d, zeros840, deg_mode=True)
    zs1, dinv = _tc_a(xp, w1p, dega)
    agg1 = _aggregate(zs1, src2d, dst2d, zeros840)
    zs2 = _tc_b(agg1, zs1, dinv, w2p, b1p.reshape(1, -1))
    agg2 = _aggregate(zs2, src2d, dst2d, zeros840)
    zs3 = _tc_b(agg2, zs2, dinv, W3, b2.reshape(1, -1))
    agg3 = _aggregate(zs3, src2d, dst2d, zeros840)

    batchr = batch.reshape(_NBLK, 1, _BN)
    return _tc_c(agg3, zs3, dinv, b3.reshape(1, -1), batchr,
                 Wfc1, bfc1.reshape(1, -1), Wfc2, bfc2.reshape(1, -1))


# speedup vs baseline: 1.4774x; 1.0428x over previous
import jax
import jax.numpy as jnp
from jax import lax
from jax.experimental import pallas as pl
from jax.experimental.pallas import tpu as pltpu
from jax.experimental.pallas import tpu_sc as plsc

F32 = jnp.float32

_N = 100000
_E = 1600000
_G = 64

_CH = 8960
_NCHUNK = 12
_NPAD = _CH * _NCHUNK
_EPT = 50176
_EPAD = _EPT * 32
_ER = _EPAD // 128
_DR = 784

_BN = 2000
_NBLK = _N // _BN


def _agg_body(deg_mode, zs_hbm, src_hbm, dst_hbm, zeros_hbm, out_hbm,
              sbuf, dbuf, csrc, coff, rows, sstage, ostage, acc,
              sem_g, sem_s, sem_l):
    cid = lax.axis_index("c")
    sid = lax.axis_index("s")

    def _drain(sem):
        pltpu.make_async_copy(zs_hbm.at[sstage.at[0]],
                              rows.at[pl.ds(0, 128)], sem).wait()

    def _drain_l():
        pltpu.make_async_copy(dst_hbm.at[pl.ds(0, 16)],
                              dbuf.at[pl.ds(0, 16)], sem_l).wait()

    if deg_mode:
        pltpu.sync_copy(zs_hbm, rows.at[pl.ds(0, 128)])

    def flush(b, g):
        slot = jnp.bitwise_and(g, 1)
        oslot = jnp.bitwise_and(g + 1, 1)

        @pl.when(g >= 2)
        def _():
            _drain(sem_s)

        for q in range(8):
            if not deg_mode:
                sstage[slot, pl.ds(q * 16, 16)] = csrc[pl.ds(b * 128 + q * 16, 16)]
            ostage[slot, pl.ds(q * 16, 16)] = coff[pl.ds(b * 128 + q * 16, 16)]
        if deg_mode:
            pltpu.async_copy(rows.at[pl.ds(0, 128)],
                             acc.at[ostage.at[slot]], sem_s, add=True)
        else:
            pltpu.async_copy(zs_hbm.at[sstage.at[slot]],
                             rows.at[pl.ds(slot * 128, 128)], sem_g)

            @pl.when(g >= 1)
            def _():
                _drain(sem_g)
                pltpu.async_copy(rows.at[pl.ds(oslot * 128, 128)],
                                 acc.at[ostage.at[oslot]], sem_s, add=True)
        return g + 1

    def chunk_body(kk, carry):
        chunk = 2 * kk + cid
        lo = chunk * _CH
        hi = lo + _CH
        r0 = sid * 784
        if not deg_mode:
            pltpu.async_copy(src_hbm.at[pl.ds(r0, 16)],
                             sbuf.at[pl.ds(0, 16)], sem_l)
        pltpu.async_copy(dst_hbm.at[pl.ds(r0, 16)],
                         dbuf.at[pl.ds(0, 16)], sem_l)
        pltpu.sync_copy(zeros_hbm.at[pl.ds(0, 560)],
                        acc.at[pl.ds(sid * 560, 560)])
        plsc.subcore_barrier()

        def block_body(u, st):
            cur, g = st
            p16 = jnp.bitwise_and(u, 1) * 16
            np16 = jnp.bitwise_and(u + 1, 1) * 16
            if not deg_mode:
                _drain_l()
            _drain_l()

            @pl.when(u + 1 < 49)
            def _():
                r1 = sid * 784 + (u + 1) * 16
                if not deg_mode:
                    pltpu.async_copy(src_hbm.at[pl.ds(r1, 16)],
                                     sbuf.at[pl.ds(np16, 16)], sem_l)
                pltpu.async_copy(dst_hbm.at[pl.ds(r1, 16)],
                                 dbuf.at[pl.ds(np16, 16)], sem_l)

            for j in range(16):
                for q in range(8):
                    d = dbuf[p16 + j, pl.ds(q * 16, 16)]
                    off = d - lo
                    mask = plsc.bitcast(off, jnp.uint32) < jnp.uint32(_CH)
                    plsc.store_compressed(coff.at[pl.ds(cur, 16)],
                                          off, mask=mask)
                    if not deg_mode:
                        s = sbuf[p16 + j, pl.ds(q * 16, 16)]
                        plsc.store_compressed(csrc.at[pl.ds(cur, 16)],
                                              s, mask=mask)
                    cur = cur + plsc.all_reduce_population_count(mask)[0]
            nb = lax.shift_right_logical(cur, 7)
            g = lax.fori_loop(0, nb, flush, g)
            base = nb * 128
            for q in range(8):
                coff[pl.ds(q * 16, 16)] = coff[pl.ds(base + q * 16, 16)]
                if not deg_mode:
                    csrc[pl.ds(q * 16, 16)] = csrc[pl.ds(base + q * 16, 16)]
            return (cur - base, g)

        cur, g = lax.fori_loop(0, 49, block_body, (0, 0))
        pad_off = jnp.full((16,), _CH, jnp.int32)
        pad_src = jnp.zeros((16,), jnp.int32)
        for q in range(8):
            coff[pl.ds(cur + q * 16, 16)] = pad_off
            if not deg_mode:
                csrc[pl.ds(cur + q * 16, 16)] = pad_src
        nbf = lax.shift_right_logical(cur + 127, 7)
        g = lax.fori_loop(0, nbf, flush, g)

        if not deg_mode:
            @pl.when(g >= 1)
            def _():
                _drain(sem_g)
                lslot = jnp.bitwise_and(g + 1, 1)
                pltpu.async_copy(rows.at[pl.ds(lslot * 128, 128)],
                                 acc.at[ostage.at[lslot]], sem_s, add=True)

        @pl.when(g >= 1)
        def _():
            _drain(sem_s)

        @pl.when(g >= 2)
        def _():
            _drain(sem_s)

        plsc.subcore_barrier()
        r = sid * 560
        pltpu.sync_copy(acc.at[pl.ds(r, 560)],
                        out_hbm.at[pl.ds(lo + r, 560)])
        plsc.subcore_barrier()
        return carry

    lax.fori_loop(0, _NCHUNK // 2, chunk_body, 0)


def _aggregate(zs, src2d, dst2d, zeros840, deg_mode=False):
    import functools as _ft
    return pl.kernel(
        _ft.partial(_agg_body, deg_mode),
        out_type=jax.ShapeDtypeStruct((_NPAD, 128), F32),
        mesh=plsc.VectorSubcoreMesh(core_axis_name="c", subcore_axis_name="s"),
        compiler_params=pltpu.CompilerParams(needs_layout_passes=False),
        scratch_types=[
            pltpu.VMEM((32, 128), jnp.int32),
            pltpu.VMEM((32, 128), jnp.int32),
            pltpu.VMEM((2304,), jnp.int32),
            pltpu.VMEM((2304,), jnp.int32),
            pltpu.VMEM((256, 128), F32),
            pltpu.VMEM((2, 128), jnp.int32),
            pltpu.VMEM((2, 128), jnp.int32),
            pltpu.VMEM_SHARED((_CH + 16, 128), F32),
            pltpu.SemaphoreType.DMA,
            pltpu.SemaphoreType.DMA,
            pltpu.SemaphoreType.DMA,
        ],
    )(zs, src2d, dst2d, zeros840)


def _tc_a_body(xp_ref, w1_ref, dega_ref, zs_ref, dinv_ref):
    dv = lax.rsqrt(dega_ref[:, 0:1] + 1.0)
    dinv_ref[...] = dv
    z = jnp.dot(xp_ref[...], w1_ref[...], preferred_element_type=F32)
    zs_ref[...] = z * dv


def _tc_a(xp, w1p, dega):
    return pl.pallas_call(
        _tc_a_body,
        grid=(_NBLK,),
        in_specs=[
            pl.BlockSpec((_BN, 128), lambda i: (i, 0)),
            pl.BlockSpec((128, 128), lambda i: (0, 0)),
            pl.BlockSpec((_BN, 128), lambda i: (i, 0)),
        ],
        out_specs=[
            pl.BlockSpec((_BN, 128), lambda i: (i, 0)),
            pl.BlockSpec((_BN, 1), lambda i: (i, 0)),
        ],
        out_shape=[
            jax.ShapeDtypeStruct((_N, 128), F32),
            jax.ShapeDtypeStruct((_N, 1), F32),
        ],
    )(xp, w1p, dega)


def _tc_b_body(agg_ref, zs_ref, dinv_ref, w_ref, b_ref, out_ref):
    dv = dinv_ref[...]
    h = jax.nn.relu(dv * (agg_ref[...] + zs_ref[...]) + b_ref[...])
    out_ref[...] = jnp.dot(h, w_ref[...], preferred_element_type=F32) * dv


def _tc_b(agg, zs, dinv, w, b2d):
    return pl.pallas_call(
        _tc_b_body,
        grid=(_NBLK,),
        in_specs=[
            pl.BlockSpec((_BN, 128), lambda i: (i, 0)),
            pl.BlockSpec((_BN, 128), lambda i: (i, 0)),
            pl.BlockSpec((_BN, 1), lambda i: (i, 0)),
            pl.BlockSpec((128, 128), lambda i: (0, 0)),
            pl.BlockSpec((1, 128), lambda i: (0, 0)),
        ],
        out_specs=pl.BlockSpec((_BN, 128), lambda i: (i, 0)),
        out_shape=jax.ShapeDtypeStruct((_N, 128), F32),
    )(agg, zs, dinv, w, b2d)


def _tc_c_body(agg_ref, zs_ref, dinv_ref, b3_ref, batch_ref,
               wfc1_ref, bfc1_ref, wfc2_ref, bfc2_ref, out_ref,
               sums, cnts):
    i = pl.program_id(0)

    @pl.when(i == 0)
    def _():
        sums[...] = jnp.zeros_like(sums)
        cnts[...] = jnp.zeros_like(cnts)

    dv = dinv_ref[...]
    h = jax.nn.relu(dv * (agg_ref[...] + zs_ref[...]) + b3_ref[...])
    bt = batch_ref[0, 0, :]
    gid = lax.broadcasted_iota(jnp.int32, (_G, _BN), 0)
    mask = (bt[None, :] == gid).astype(F32)
    sums[...] += jnp.dot(mask, h, preferred_element_type=F32)
    cnts[...] += jnp.sum(mask, axis=1, keepdims=True)

    @pl.when(i == _NBLK - 1)
    def _():
        g = sums[...] / jnp.maximum(cnts[...], 1.0)
        a = jax.nn.relu(
            jnp.dot(g, wfc1_ref[...], preferred_element_type=F32)
            + bfc1_ref[...])
        out_ref[...] = (
            jnp.dot(a, wfc2_ref[...], preferred_element_type=F32)
            + bfc2_ref[...])


def _tc_c(agg, zs, dinv, b3, batchr, wfc1, bfc1, wfc2, bfc2):
    return pl.pallas_call(
        _tc_c_body,
        grid=(_NBLK,),
        in_specs=[
            pl.BlockSpec((_BN, 128), lambda i: (i, 0)),
            pl.BlockSpec((_BN, 128), lambda i: (i, 0)),
            pl.BlockSpec((_BN, 1), lambda i: (i, 0)),
            pl.BlockSpec((1, 128), lambda i: (0, 0)),
            pl.BlockSpec((1, 1, _BN), lambda i: (i, 0, 0)),
            pl.BlockSpec((128, 64), lambda i: (0, 0)),
            pl.BlockSpec((1, 64), lambda i: (0, 0)),
            pl.BlockSpec((64, 10), lambda i: (0, 0)),
            pl.BlockSpec((1, 10), lambda i: (0, 0)),
        ],
        out_specs=pl.BlockSpec((_G, 10), lambda i: (0, 0)),
        out_shape=jax.ShapeDtypeStruct((_G, 10), F32),
        scratch_shapes=[
            pltpu.VMEM((_G, 128), F32),
            pltpu.VMEM((_G, 1), F32),
        ],
    )(agg, zs, dinv, b3, batchr, wfc1, bfc1, wfc2, bfc2)


def kernel(x, edge_index, batch, W1, b1, W2, b2, W3, b3,
           Wfc1, bfc1, Wfc2, bfc2):
    pad = _EPAD - _E
    src2d = jnp.pad(edge_index[0], (0, pad)).reshape(_ER, 128)
    dst2d = jnp.pad(edge_index[1], (0, pad),
                    constant_values=_N).reshape(_ER, 128)

    zeros840 = jnp.zeros((840, 128), F32)

    xp = jnp.pad(x, ((0, 0), (0, 128 - x.shape[1])))
    w1p = jnp.pad(W1, ((0, 128 - W1.shape[0]), (0, 128 - W1.shape[1])))
    w2p = jnp.pad(W2, ((0, 64), (0, 0)))
    b1p = jnp.pad(b1, (0, 64))

    onehot = jnp.zeros((128, 128), F32).at[:, 0].set(1.0)
    dega = _aggregate(onehot, src2d, dst2d, zeros840, deg_mode=True)
    zs1, dinv = _tc_a(xp, w1p, dega)
    agg1 = _aggregate(zs1, src2d, dst2d, zeros840)
    zs2 = _tc_b(agg1, zs1, dinv, w2p, b1p.reshape(1, -1))
    agg2 = _aggregate(zs2, src2d, dst2d, zeros840)
    zs3 = _tc_b(agg2, zs2, dinv, W3, b2.reshape(1, -1))
    agg3 = _aggregate(zs3, src2d, dst2d, zeros840)

    batchr = batch.reshape(_NBLK, 1, _BN)
    return _tc_c(agg3, zs3, dinv, b3.reshape(1, -1), batchr,
                 Wfc1, bfc1.reshape(1, -1), Wfc2, bfc2.reshape(1, -1))


# speedup vs baseline: 1.5770x; 1.0674x over previous
import jax
import jax.numpy as jnp
from jax import lax
from jax.experimental import pallas as pl
from jax.experimental.pallas import tpu as pltpu
from jax.experimental.pallas import tpu_sc as plsc

F32 = jnp.float32

_N = 100000
_E = 1600000
_G = 64

_CH = 10496
_NCHUNK = 10
_CHD = 12928
_NCHUNKD = 8
_EPT = 50176
_EPAD = _EPT * 32
_ER = _EPAD // 128
_DR = 784

_BN = 2000
_NBLK = _N // _BN


def _agg_body(deg_mode, zs_hbm, src_hbm, dst_hbm, zeros_hbm, out_hbm,
              sbuf, dbuf, csrc, coff, rows, sstage, ostage, acc,
              sem_g, sem_s, sem_l):
    ch = _CHD if deg_mode else _CH
    nch = _NCHUNKD if deg_mode else _NCHUNK
    rpt = ch // 16
    cid = lax.axis_index("c")
    sid = lax.axis_index("s")

    def _drain(sem):
        pltpu.make_async_copy(zs_hbm.at[sstage.at[0]],
                              rows.at[pl.ds(0, 128)], sem).wait()

    def _drain_l():
        pltpu.make_async_copy(dst_hbm.at[pl.ds(0, 16)],
                              dbuf.at[pl.ds(0, 16)], sem_l).wait()

    if deg_mode:
        pltpu.sync_copy(zs_hbm, rows.at[pl.ds(0, 128)])

    def flush(b, g):
        slot = jnp.bitwise_and(g, 1)
        oslot = jnp.bitwise_and(g + 1, 1)

        @pl.when(g >= 2)
        def _():
            _drain(sem_s)

        for q in range(8):
            if not deg_mode:
                sstage[slot, pl.ds(q * 16, 16)] = csrc[pl.ds(b * 128 + q * 16, 16)]
            ostage[slot, pl.ds(q * 16, 16)] = coff[pl.ds(b * 128 + q * 16, 16)]
        if deg_mode:
            pltpu.async_copy(rows.at[pl.ds(0, 128)],
                             acc.at[ostage.at[slot]], sem_s, add=True)
        else:
            pltpu.async_copy(zs_hbm.at[sstage.at[slot]],
                             rows.at[pl.ds(slot * 128, 128)], sem_g)

            @pl.when(g >= 1)
            def _():
                _drain(sem_g)
                pltpu.async_copy(rows.at[pl.ds(oslot * 128, 128)],
                                 acc.at[ostage.at[oslot]], sem_s, add=True)
        return g + 1

    def chunk_body(kk, carry):
        chunk = 2 * kk + cid
        lo = chunk * ch
        hi = lo + ch
        r0 = sid * 784
        if not deg_mode:
            pltpu.async_copy(src_hbm.at[pl.ds(r0, 16)],
                             sbuf.at[pl.ds(0, 16)], sem_l)
        pltpu.async_copy(dst_hbm.at[pl.ds(r0, 16)],
                         dbuf.at[pl.ds(0, 16)], sem_l)
        pltpu.sync_copy(zeros_hbm.at[pl.ds(0, rpt)],
                        acc.at[pl.ds(sid * rpt, rpt)])
        plsc.subcore_barrier()

        def block_body(u, st):
            cur, g = st
            p16 = jnp.bitwise_and(u, 1) * 16
            np16 = jnp.bitwise_and(u + 1, 1) * 16
            if not deg_mode:
                _drain_l()
            _drain_l()

            @pl.when(u + 1 < 49)
            def _():
                r1 = sid * 784 + (u + 1) * 16
                if not deg_mode:
                    pltpu.async_copy(src_hbm.at[pl.ds(r1, 16)],
                                     sbuf.at[pl.ds(np16, 16)], sem_l)
                pltpu.async_copy(dst_hbm.at[pl.ds(r1, 16)],
                                 dbuf.at[pl.ds(np16, 16)], sem_l)

            for j in range(16):
                for q in range(8):
                    d = dbuf[p16 + j, pl.ds(q * 16, 16)]
                    off = d - lo
                    mask = plsc.bitcast(off, jnp.uint32) < jnp.uint32(ch)
                    plsc.store_compressed(coff.at[pl.ds(cur, 16)],
                                          off, mask=mask)
                    if not deg_mode:
                        s = sbuf[p16 + j, pl.ds(q * 16, 16)]
                        plsc.store_compressed(csrc.at[pl.ds(cur, 16)],
                                              s, mask=mask)
                    cur = cur + plsc.all_reduce_population_count(mask)[0]
            nb = lax.shift_right_logical(cur, 7)
            g = lax.fori_loop(0, nb, flush, g)
            base = nb * 128
            for q in range(8):
                coff[pl.ds(q * 16, 16)] = coff[pl.ds(base + q * 16, 16)]
                if not deg_mode:
                    csrc[pl.ds(q * 16, 16)] = csrc[pl.ds(base + q * 16, 16)]
            return (cur - base, g)

        cur, g = lax.fori_loop(0, 49, block_body, (0, 0))
        pad_off = jnp.full((16,), ch, jnp.int32)
        pad_src = jnp.zeros((16,), jnp.int32)
        for q in range(8):
            coff[pl.ds(cur + q * 16, 16)] = pad_off
            if not deg_mode:
                csrc[pl.ds(cur + q * 16, 16)] = pad_src
        nbf = lax.shift_right_logical(cur + 127, 7)
        g = lax.fori_loop(0, nbf, flush, g)

        if not deg_mode:
            @pl.when(g >= 1)
            def _():
                _drain(sem_g)
                lslot = jnp.bitwise_and(g + 1, 1)
                pltpu.async_copy(rows.at[pl.ds(lslot * 128, 128)],
                                 acc.at[ostage.at[lslot]], sem_s, add=True)

        @pl.when(g >= 1)
        def _():
            _drain(sem_s)

        @pl.when(g >= 2)
        def _():
            _drain(sem_s)

        plsc.subcore_barrier()
        r = sid * rpt
        pltpu.sync_copy(acc.at[pl.ds(r, rpt)],
                        out_hbm.at[pl.ds(lo + r, rpt)])
        plsc.subcore_barrier()
        return carry

    lax.fori_loop(0, nch // 2, chunk_body, 0)


def _aggregate(zs, src2d, dst2d, zeros840, deg_mode=False):
    import functools as _ft
    if deg_mode:
        ch, nch = _CHD, _NCHUNKD
        sbuf_t = pltpu.VMEM((8, 128), jnp.int32)
        csrc_t = pltpu.VMEM((16,), jnp.int32)
        rows_t = pltpu.VMEM((128, 128), F32)
    else:
        ch, nch = _CH, _NCHUNK
        sbuf_t = pltpu.VMEM((32, 128), jnp.int32)
        csrc_t = pltpu.VMEM((2304,), jnp.int32)
        rows_t = pltpu.VMEM((256, 128), F32)
    return pl.kernel(
        _ft.partial(_agg_body, deg_mode),
        out_type=jax.ShapeDtypeStruct((ch * nch, 128), F32),
        mesh=plsc.VectorSubcoreMesh(core_axis_name="c", subcore_axis_name="s"),
        compiler_params=pltpu.CompilerParams(needs_layout_passes=False),
        scratch_types=[
            sbuf_t,
            pltpu.VMEM((32, 128), jnp.int32),
            csrc_t,
            pltpu.VMEM((2304,), jnp.int32),
            rows_t,
            pltpu.VMEM((2, 128), jnp.int32),
            pltpu.VMEM((2, 128), jnp.int32),
            pltpu.VMEM_SHARED((ch + 16, 128), F32),
            pltpu.SemaphoreType.DMA,
            pltpu.SemaphoreType.DMA,
            pltpu.SemaphoreType.DMA,
        ],
    )(zs, src2d, dst2d, zeros840)


def _tc_a_body(xp_ref, w1_ref, dega_ref, zs_ref, dinv_ref):
    dv = lax.rsqrt(dega_ref[:, 0:1] + 1.0)
    dinv_ref[...] = dv
    z = jnp.dot(xp_ref[...], w1_ref[...], preferred_element_type=F32)
    zs_ref[...] = z * dv


def _tc_a(xp, w1p, dega):
    return pl.pallas_call(
        _tc_a_body,
        grid=(_NBLK,),
        in_specs=[
            pl.BlockSpec((_BN, 128), lambda i: (i, 0)),
            pl.BlockSpec((128, 128), lambda i: (0, 0)),
            pl.BlockSpec((_BN, 128), lambda i: (i, 0)),
        ],
        out_specs=[
            pl.BlockSpec((_BN, 128), lambda i: (i, 0)),
            pl.BlockSpec((_BN, 1), lambda i: (i, 0)),
        ],
        out_shape=[
            jax.ShapeDtypeStruct((_N, 128), F32),
            jax.ShapeDtypeStruct((_N, 1), F32),
        ],
    )(xp, w1p, dega)


def _tc_b_body(agg_ref, zs_ref, dinv_ref, w_ref, b_ref, out_ref):
    dv = dinv_ref[...]
    h = jax.nn.relu(dv * (agg_ref[...] + zs_ref[...]) + b_ref[...])
    out_ref[...] = jnp.dot(h, w_ref[...], preferred_element_type=F32) * dv


def _tc_b(agg, zs, dinv, w, b2d):
    return pl.pallas_call(
        _tc_b_body,
        grid=(_NBLK,),
        in_specs=[
            pl.BlockSpec((_BN, 128), lambda i: (i, 0)),
            pl.BlockSpec((_BN, 128), lambda i: (i, 0)),
            pl.BlockSpec((_BN, 1), lambda i: (i, 0)),
            pl.BlockSpec((128, 128), lambda i: (0, 0)),
            pl.BlockSpec((1, 128), lambda i: (0, 0)),
        ],
        out_specs=pl.BlockSpec((_BN, 128), lambda i: (i, 0)),
        out_shape=jax.ShapeDtypeStruct((_N, 128), F32),
    )(agg, zs, dinv, w, b2d)


def _tc_c_body(agg_ref, zs_ref, dinv_ref, b3_ref, batch_ref,
               wfc1_ref, bfc1_ref, wfc2_ref, bfc2_ref, out_ref,
               sums, cnts):
    i = pl.program_id(0)

    @pl.when(i == 0)
    def _():
        sums[...] = jnp.zeros_like(sums)
        cnts[...] = jnp.zeros_like(cnts)

    dv = dinv_ref[...]
    h = jax.nn.relu(dv * (agg_ref[...] + zs_ref[...]) + b3_ref[...])
    bt = batch_ref[0, 0, :]
    gid = lax.broadcasted_iota(jnp.int32, (_G, _BN), 0)
    mask = (bt[None, :] == gid).astype(F32)
    sums[...] += jnp.dot(mask, h, preferred_element_type=F32)
    cnts[...] += jnp.sum(mask, axis=1, keepdims=True)

    @pl.when(i == _NBLK - 1)
    def _():
        g = sums[...] / jnp.maximum(cnts[...], 1.0)
        a = jax.nn.relu(
            jnp.dot(g, wfc1_ref[...], preferred_element_type=F32)
            + bfc1_ref[...])
        out_ref[...] = (
            jnp.dot(a, wfc2_ref[...], preferred_element_type=F32)
            + bfc2_ref[...])


def _tc_c(agg, zs, dinv, b3, batchr, wfc1, bfc1, wfc2, bfc2):
    return pl.pallas_call(
        _tc_c_body,
        grid=(_NBLK,),
        in_specs=[
            pl.BlockSpec((_BN, 128), lambda i: (i, 0)),
            pl.BlockSpec((_BN, 128), lambda i: (i, 0)),
            pl.BlockSpec((_BN, 1), lambda i: (i, 0)),
            pl.BlockSpec((1, 128), lambda i: (0, 0)),
            pl.BlockSpec((1, 1, _BN), lambda i: (i, 0, 0)),
            pl.BlockSpec((128, 64), lambda i: (0, 0)),
            pl.BlockSpec((1, 64), lambda i: (0, 0)),
            pl.BlockSpec((64, 10), lambda i: (0, 0)),
            pl.BlockSpec((1, 10), lambda i: (0, 0)),
        ],
        out_specs=pl.BlockSpec((_G, 10), lambda i: (0, 0)),
        out_shape=jax.ShapeDtypeStruct((_G, 10), F32),
        scratch_shapes=[
            pltpu.VMEM((_G, 128), F32),
            pltpu.VMEM((_G, 1), F32),
        ],
    )(agg, zs, dinv, b3, batchr, wfc1, bfc1, wfc2, bfc2)


def kernel(x, edge_index, batch, W1, b1, W2, b2, W3, b3,
           Wfc1, bfc1, Wfc2, bfc2):
    pad = _EPAD - _E
    src2d = jnp.pad(edge_index[0], (0, pad)).reshape(_ER, 128)
    dst2d = jnp.pad(edge_index[1], (0, pad),
                    constant_values=_N).reshape(_ER, 128)

    zeros840 = jnp.zeros((840, 128), F32)

    xp = jnp.pad(x, ((0, 0), (0, 128 - x.shape[1])))
    w1p = jnp.pad(W1, ((0, 128 - W1.shape[0]), (0, 128 - W1.shape[1])))
    w2p = jnp.pad(W2, ((0, 64), (0, 0)))
    b1p = jnp.pad(b1, (0, 64))

    onehot = jnp.zeros((128, 128), F32).at[:, 0].set(1.0)
    dega = _aggregate(onehot, src2d, dst2d, zeros840, deg_mode=True)
    zs1, dinv = _tc_a(xp, w1p, dega)
    agg1 = _aggregate(zs1, src2d, dst2d, zeros840)
    zs2 = _tc_b(agg1, zs1, dinv, w2p, b1p.reshape(1, -1))
    agg2 = _aggregate(zs2, src2d, dst2d, zeros840)
    zs3 = _tc_b(agg2, zs2, dinv, W3, b2.reshape(1, -1))
    agg3 = _aggregate(zs3, src2d, dst2d, zeros840)

    batchr = batch.reshape(_NBLK, 1, _BN)
    return _tc_c(agg3, zs3, dinv, b3.reshape(1, -1), batchr,
                 Wfc1, bfc1.reshape(1, -1), Wfc2, bfc2.reshape(1, -1))
